# baseline (speedup 1.0000x reference)
import functools

import jax
import jax.numpy as jnp
from jax import lax
from jax.experimental import pallas as pl
from jax.experimental.pallas import tpu as pltpu
from jax.experimental.pallas import tpu_sc as plsc

N = 10000
D = 128
E = 320000

NUM_TILES = 16
NUM_CORES = 2
NW = NUM_TILES * NUM_CORES
C = 128

NP = 10240
CHUNKS = 80
EP = NW * CHUNKS * C
EDGES_PER_TILE = EP // NW
NBUF = 4
STRIPE = NP // NUM_TILES

ROW_BLK = 1024
TC_GRID = NP // ROW_BLK


def _k1_body(f_ref, w1_ref, b1_ref, watt_ref, battr_ref, at_ref, msg_ref, pq_ref):
    fb = f_ref[...]
    msg_ref[...] = (
        jnp.dot(fb, w1_ref[...].T, preferred_element_type=jnp.float32) + b1_ref[...]
    )
    v = jnp.dot(at_ref[...], watt_ref[...], preferred_element_type=jnp.float32)
    c = jnp.dot(at_ref[...], battr_ref[...].T, preferred_element_type=jnp.float32)
    pq_ref[...] = jnp.dot(v, fb.T, preferred_element_type=jnp.float32) + c


def _k1(f_pad, W1_w, W1_b, Watt_w, Watt_b, aT):
    return pl.pallas_call(
        _k1_body,
        grid=(TC_GRID,),
        in_specs=[
            pl.BlockSpec((ROW_BLK, D), lambda i: (i, 0)),
            pl.BlockSpec((D, D), lambda i: (0, 0)),
            pl.BlockSpec((1, D), lambda i: (0, 0)),
            pl.BlockSpec((D, D), lambda i: (0, 0)),
            pl.BlockSpec((1, D), lambda i: (0, 0)),
            pl.BlockSpec((2, D), lambda i: (0, 0)),
        ],
        out_specs=[
            pl.BlockSpec((ROW_BLK, D), lambda i: (i, 0)),
            pl.BlockSpec((2, ROW_BLK), lambda i: (0, i)),
        ],
        out_shape=[
            jax.ShapeDtypeStruct((NP, D), jnp.float32),
            jax.ShapeDtypeStruct((2, NP), jnp.float32),
        ],
    )(f_pad, W1_w, W1_b.reshape(1, D), Watt_w, Watt_b.reshape(1, D), aT)


def _k2_body(srcr_hbm, dstr_hbm, p_hbm, q_hbm, e_hbm, den_hbm,
             siv, div, pg, qg, ev, zb, den_sh, sem):
    cid = lax.axis_index("c")
    sid = lax.axis_index("s")
    wid = sid * NUM_CORES + cid

    def _z(i, carry):
        zb[pl.ds(i * 16, 16)] = jnp.zeros((16,), jnp.float32)
        return carry
    lax.fori_loop(0, STRIPE // 16, _z, 0)
    pltpu.sync_copy(zb, den_sh.at[pl.ds(sid * STRIPE, STRIPE)])
    plsc.subcore_barrier()

    rbase = wid * (EDGES_PER_TILE // C)

    def _chunk(t, carry):
        rb = rbase + t * 2
        pltpu.sync_copy(srcr_hbm.at[pl.ds(rb, 2), :], siv)
        pltpu.sync_copy(dstr_hbm.at[pl.ds(rb, 2), :], div)
        cps = [
            pltpu.async_copy(p_hbm.at[siv.at[0]], pg.at[0], sem),
            pltpu.async_copy(q_hbm.at[div.at[0]], qg.at[0], sem),
            pltpu.async_copy(p_hbm.at[siv.at[1]], pg.at[1], sem),
            pltpu.async_copy(q_hbm.at[div.at[1]], qg.at[1], sem),
        ]
        for cp in cps:
            cp.wait()
        for a in range(2):
            for j in range(C // 16):
                x = pg[a, pl.ds(j * 16, 16)] + qg[a, pl.ds(j * 16, 16)]
                x = jnp.where(x >= 0.0, x, x * 0.01)
                ev[a, pl.ds(j * 16, 16)] = jnp.exp(x)
        pltpu.sync_copy(ev, e_hbm.at[pl.ds(rb, 2), :])
        pltpu.sync_copy(ev.at[0], den_sh.at[div.at[0]], add=True)
        pltpu.sync_copy(ev.at[1], den_sh.at[div.at[1]], add=True)
        return carry

    lax.fori_loop(0, CHUNKS // 2, _chunk, 0)
    plsc.subcore_barrier()

    pltpu.sync_copy(den_sh.at[pl.ds(sid * STRIPE, STRIPE)], zb)
    pltpu.sync_copy(zb, den_hbm.at[cid, pl.ds(sid * STRIPE, STRIPE)])


def _k2(srcr, dstr, p, q):
    mesh = plsc.VectorSubcoreMesh(core_axis_name="c", subcore_axis_name="s")
    return pl.kernel(
        _k2_body,
        mesh=mesh,
        out_type=[
            jax.ShapeDtypeStruct((EP // C, C), jnp.float32),
            jax.ShapeDtypeStruct((2, NP), jnp.float32),
        ],
        scratch_types=[
            pltpu.VMEM((2, C), jnp.int32),
            pltpu.VMEM((2, C), jnp.int32),
            pltpu.VMEM((2, C), jnp.float32),
            pltpu.VMEM((2, C), jnp.float32),
            pltpu.VMEM((2, C), jnp.float32),
            pltpu.VMEM((STRIPE,), jnp.float32),
            pltpu.VMEM_SHARED((NP,), jnp.float32),
            pltpu.SemaphoreType.DMA,
        ],
    )(srcr, dstr, p, q)


def _k3_body(msg_ref, den_ref, out_ref):
    d = den_ref[0, :] + den_ref[1, :] + 1e-9
    out_ref[...] = msg_ref[...] / d[:, None]


def _k3(msg, den):
    return pl.pallas_call(
        _k3_body,
        grid=(TC_GRID,),
        in_specs=[
            pl.BlockSpec((ROW_BLK, D), lambda i: (i, 0)),
            pl.BlockSpec((2, ROW_BLK), lambda i: (0, i)),
        ],
        out_specs=pl.BlockSpec((ROW_BLK, D), lambda i: (i, 0)),
        out_shape=jax.ShapeDtypeStruct((NP, D), jnp.float32),
    )(msg, den)


def _k4_body(srcr_hbm, dstr_hbm, e_hbm, msg2_hbm, hp_hbm,
             siv, div, ev, rows, acc_sh, sem):
    cid = lax.axis_index("c")
    sid = lax.axis_index("s")
    wid = sid * NUM_CORES + cid

    def _z(i, carry):
        for a in range(2):
            for j in range(D // 16):
                rows[a, i, pl.ds(j * 16, 16)] = jnp.zeros((16,), jnp.float32)
        return carry
    lax.fori_loop(0, C, _z, 0)
    for r in range(STRIPE // C):
        pltpu.sync_copy(
            rows.at[0], acc_sh.at[pl.ds(sid * STRIPE + r * C, C), :])
    plsc.subcore_barrier()

    rbase = wid * (EDGES_PER_TILE // C)

    def _chunk(t, carry):
        rb = rbase + t * 2
        pltpu.sync_copy(srcr_hbm.at[pl.ds(rb, 2), :], siv)
        pltpu.sync_copy(dstr_hbm.at[pl.ds(rb, 2), :], div)
        pltpu.sync_copy(e_hbm.at[pl.ds(rb, 2), :], ev)
        g0 = pltpu.async_copy(msg2_hbm.at[div.at[0]], rows.at[0], sem)
        g1 = pltpu.async_copy(msg2_hbm.at[div.at[1]], rows.at[1], sem)
        g0.wait()
        g1.wait()

        for a in range(2):
            def _grp(g2, c2, _a=a):
                ev16 = ev[_a, pl.ds(g2 * 16, 16)]
                for l in range(16):
                    e = ev16[l]
                    i = g2 * 16 + l
                    for j in range(D // 16):
                        rows[_a, i, pl.ds(j * 16, 16)] = (
                            rows[_a, i, pl.ds(j * 16, 16)] * e)
                return c2
            lax.fori_loop(0, C // 16, _grp, 0)

        pltpu.sync_copy(rows.at[0], acc_sh.at[siv.at[0]], add=True)
        pltpu.sync_copy(rows.at[1], acc_sh.at[siv.at[1]], add=True)
        return carry

    lax.fori_loop(0, CHUNKS // 2, _chunk, 0)
    plsc.subcore_barrier()

    for r in range(STRIPE // C):
        pltpu.sync_copy(
            acc_sh.at[pl.ds(sid * STRIPE + r * C, C), :], rows.at[0])
        pltpu.sync_copy(
            rows.at[0], hp_hbm.at[cid, pl.ds(sid * STRIPE + r * C, C), :])


def _k4(srcr, dstr, e_edge, msg2):
    mesh = plsc.VectorSubcoreMesh(core_axis_name="c", subcore_axis_name="s")
    return pl.kernel(
        _k4_body,
        mesh=mesh,
        out_type=jax.ShapeDtypeStruct((2, NP, D), jnp.float32),
        scratch_types=[
            pltpu.VMEM((2, C), jnp.int32),
            pltpu.VMEM((2, C), jnp.int32),
            pltpu.VMEM((2, C), jnp.float32),
            pltpu.VMEM((2, C, D), jnp.float32),
            pltpu.VMEM_SHARED((NP, D), jnp.float32),
            pltpu.SemaphoreType.DMA,
        ],
    )(srcr, dstr, e_edge, msg2)


def _k5_body(f_ref, hp_ref, w2_ref, b2_ref, out_ref):
    fb = f_ref[...]
    hn = hp_ref[0] + hp_ref[1]
    t2 = jnp.dot(fb * hn, w2_ref[...].T, preferred_element_type=jnp.float32) + b2_ref[...]
    o = fb + hn + t2
    out_ref[...] = jnp.where(o >= 0.0, o, o * 0.01)


def _k5(f_pad, hp, W2_w, W2_b):
    return pl.pallas_call(
        _k5_body,
        grid=(TC_GRID,),
        in_specs=[
            pl.BlockSpec((ROW_BLK, D), lambda i: (i, 0)),
            pl.BlockSpec((2, ROW_BLK, D), lambda i: (0, i, 0)),
            pl.BlockSpec((D, D), lambda i: (0, 0)),
            pl.BlockSpec((1, D), lambda i: (0, 0)),
        ],
        out_specs=pl.BlockSpec((ROW_BLK, D), lambda i: (i, 0)),
        out_shape=jax.ShapeDtypeStruct((NP, D), jnp.float32),
    )(f_pad, hp, W2_w, W2_b.reshape(1, D))


@jax.jit
def _run(indices, features, W1_w, W1_b, W2_w, W2_b, Watt_w, Watt_b, a):
    f_pad = jnp.pad(features, ((0, NP - N), (0, 0)))
    idx_pad = jnp.pad(indices, ((0, 0), (0, EP - E)), constant_values=N)
    srcr = idx_pad[0].reshape(EP // C, C)
    dstr = idx_pad[1].reshape(EP // C, C)
    aT = a.reshape(2, D)

    msg, pq = _k1(f_pad, W1_w, W1_b, Watt_w, Watt_b, aT)
    e_edge, den = _k2(srcr, dstr, pq[0], pq[1])
    msg2 = _k3(msg, den)
    hp = _k4(srcr, dstr, e_edge, msg2)
    out = _k5(f_pad, hp, W2_w, W2_b)
    return out[:N]


def kernel(indices, features, num_nodes, W1_w, W1_b, W2_w, W2_b, Watt_w, Watt_b, a):
    return _run(indices, features, W1_w, W1_b, W2_w, W2_b, Watt_w, Watt_b, a)


# speedup vs baseline: 1.2636x; 1.2636x over previous
import functools

import jax
import jax.numpy as jnp
from jax import lax
from jax.experimental import pallas as pl
from jax.experimental.pallas import tpu as pltpu
from jax.experimental.pallas import tpu_sc as plsc

N = 10000
D = 128
E = 320000

NUM_TILES = 16
NUM_CORES = 2
NW = NUM_TILES * NUM_CORES
C = 128

NP = 10240
EP = ((E + NW * C - 1) // (NW * C)) * (NW * C)
EDGES_PER_TILE = EP // NW
CHUNKS = EDGES_PER_TILE // C
STRIPE = NP // NUM_TILES

ROW_BLK = 1024
TC_GRID = NP // ROW_BLK


def _k1_body(f_ref, w1_ref, b1_ref, watt_ref, battr_ref, at_ref, msg_ref, pq_ref):
    fb = f_ref[...]
    msg_ref[...] = (
        jnp.dot(fb, w1_ref[...].T, preferred_element_type=jnp.float32) + b1_ref[...]
    )
    v = jnp.dot(at_ref[...], watt_ref[...], preferred_element_type=jnp.float32)
    c = jnp.dot(at_ref[...], battr_ref[...].T, preferred_element_type=jnp.float32)
    pq_ref[...] = jnp.dot(v, fb.T, preferred_element_type=jnp.float32) + c


def _k1(f_pad, W1_w, W1_b, Watt_w, Watt_b, aT):
    return pl.pallas_call(
        _k1_body,
        grid=(TC_GRID,),
        in_specs=[
            pl.BlockSpec((ROW_BLK, D), lambda i: (i, 0)),
            pl.BlockSpec((D, D), lambda i: (0, 0)),
            pl.BlockSpec((1, D), lambda i: (0, 0)),
            pl.BlockSpec((D, D), lambda i: (0, 0)),
            pl.BlockSpec((1, D), lambda i: (0, 0)),
            pl.BlockSpec((2, D), lambda i: (0, 0)),
        ],
        out_specs=[
            pl.BlockSpec((ROW_BLK, D), lambda i: (i, 0)),
            pl.BlockSpec((2, ROW_BLK), lambda i: (0, i)),
        ],
        out_shape=[
            jax.ShapeDtypeStruct((NP, D), jnp.float32),
            jax.ShapeDtypeStruct((2, NP), jnp.float32),
        ],
    )(f_pad, W1_w, W1_b.reshape(1, D), Watt_w, Watt_b.reshape(1, D), aT)


def _k2_body(idx_hbm, p_hbm, q_hbm, e_hbm, den_hbm, iv, pg, qg, ev, zb, den_sh, sem):
    cid = lax.axis_index("c")
    sid = lax.axis_index("s")
    wid = sid * NUM_CORES + cid

    def _z(i, carry):
        zb[pl.ds(i * 16, 16)] = jnp.zeros((16,), jnp.float32)
        return carry
    lax.fori_loop(0, STRIPE // 16, _z, 0)
    pltpu.sync_copy(zb, den_sh.at[pl.ds(sid * STRIPE, STRIPE)])
    plsc.subcore_barrier()

    base = wid * EDGES_PER_TILE

    def _chunk(t, carry):
        off = base + t * C
        pltpu.sync_copy(idx_hbm.at[:, pl.ds(off, C)], iv)
        cp1 = pltpu.async_copy(p_hbm.at[iv.at[0]], pg, sem)
        cp2 = pltpu.async_copy(q_hbm.at[iv.at[1]], qg, sem)
        cp1.wait()
        cp2.wait()
        for j in range(C // 16):
            x = pg[pl.ds(j * 16, 16)] + qg[pl.ds(j * 16, 16)]
            x = jnp.where(x >= 0.0, x, x * 0.01)
            ev[pl.ds(j * 16, 16)] = jnp.exp(x)
        pltpu.sync_copy(ev, e_hbm.at[pl.ds(off, C)])
        pltpu.sync_copy(ev, den_sh.at[iv.at[1]], add=True)
        return carry

    lax.fori_loop(0, CHUNKS, _chunk, 0)
    plsc.subcore_barrier()

    pltpu.sync_copy(den_sh.at[pl.ds(sid * STRIPE, STRIPE)], zb)
    pltpu.sync_copy(zb, den_hbm.at[cid, pl.ds(sid * STRIPE, STRIPE)])


def _k2(idx_pad, p, q):
    mesh = plsc.VectorSubcoreMesh(core_axis_name="c", subcore_axis_name="s")
    return pl.kernel(
        _k2_body,
        mesh=mesh,
        out_type=[
            jax.ShapeDtypeStruct((EP,), jnp.float32),
            jax.ShapeDtypeStruct((2, NP), jnp.float32),
        ],
        scratch_types=[
            pltpu.VMEM((2, C), jnp.int32),
            pltpu.VMEM((C,), jnp.float32),
            pltpu.VMEM((C,), jnp.float32),
            pltpu.VMEM((C,), jnp.float32),
            pltpu.VMEM((STRIPE,), jnp.float32),
            pltpu.VMEM_SHARED((NP,), jnp.float32),
            pltpu.SemaphoreType.DMA,
        ],
    )(idx_pad, p, q)


def _k3_body(msg_ref, den_ref, out_ref):
    d = den_ref[0, :] + den_ref[1, :] + 1e-9
    out_ref[...] = msg_ref[...] / d[:, None]


def _k3(msg, den):
    return pl.pallas_call(
        _k3_body,
        grid=(TC_GRID,),
        in_specs=[
            pl.BlockSpec((ROW_BLK, D), lambda i: (i, 0)),
            pl.BlockSpec((2, ROW_BLK), lambda i: (0, i)),
        ],
        out_specs=pl.BlockSpec((ROW_BLK, D), lambda i: (i, 0)),
        out_shape=jax.ShapeDtypeStruct((NP, D), jnp.float32),
    )(msg, den)


def _k4_body(idx_hbm, e_hbm, msg2_hbm, hp_hbm, iv, ev, rows, zb, acc_sh, sem):
    cid = lax.axis_index("c")
    sid = lax.axis_index("s")
    wid = sid * NUM_CORES + cid

    def _z(i, carry):
        for j in range(D // 16):
            zb[i, pl.ds(j * 16, 16)] = jnp.zeros((16,), jnp.float32)
        return carry
    lax.fori_loop(0, C, _z, 0)
    for r in range(STRIPE // C):
        pltpu.sync_copy(zb, acc_sh.at[pl.ds(sid * STRIPE + r * C, C), :])
    plsc.subcore_barrier()

    base = wid * EDGES_PER_TILE

    def _chunk(t, carry):
        off = base + t * C
        pltpu.sync_copy(idx_hbm.at[:, pl.ds(off, C)], iv)
        pltpu.sync_copy(e_hbm.at[pl.ds(off, C)], ev)
        pltpu.async_copy(msg2_hbm.at[iv.at[1]], rows, sem).wait()

        def _scale(g, c2):
            ev16 = ev[pl.ds(g * 16, 16)]
            for l in range(16):
                e = ev16[l]
                i = g * 16 + l
                for j in range(D // 16):
                    rows[i, pl.ds(j * 16, 16)] = rows[i, pl.ds(j * 16, 16)] * e
            return c2
        lax.fori_loop(0, C // 16, _scale, 0)
        pltpu.sync_copy(rows, acc_sh.at[iv.at[0]], add=True)
        return carry

    lax.fori_loop(0, CHUNKS, _chunk, 0)
    plsc.subcore_barrier()

    for r in range(STRIPE // C):
        pltpu.sync_copy(acc_sh.at[pl.ds(sid * STRIPE + r * C, C), :], zb)
        pltpu.sync_copy(zb, hp_hbm.at[cid, pl.ds(sid * STRIPE + r * C, C), :])


def _k4(idx_pad, e_edge, msg2):
    mesh = plsc.VectorSubcoreMesh(core_axis_name="c", subcore_axis_name="s")
    return pl.kernel(
        _k4_body,
        mesh=mesh,
        out_type=jax.ShapeDtypeStruct((2, NP, D), jnp.float32),
        scratch_types=[
            pltpu.VMEM((2, C), jnp.int32),
            pltpu.VMEM((C,), jnp.float32),
            pltpu.VMEM((C, D), jnp.float32),
            pltpu.VMEM((C, D), jnp.float32),
            pltpu.VMEM_SHARED((NP, D), jnp.float32),
            pltpu.SemaphoreType.DMA,
        ],
    )(idx_pad, e_edge, msg2)


def _k5_body(f_ref, hp_ref, w2_ref, b2_ref, out_ref):
    fb = f_ref[...]
    hn = hp_ref[0] + hp_ref[1]
    t2 = jnp.dot(fb * hn, w2_ref[...].T, preferred_element_type=jnp.float32) + b2_ref[...]
    o = fb + hn + t2
    out_ref[...] = jnp.where(o >= 0.0, o, o * 0.01)


def _k5(f_pad, hp, W2_w, W2_b):
    return pl.pallas_call(
        _k5_body,
        grid=(TC_GRID,),
        in_specs=[
            pl.BlockSpec((ROW_BLK, D), lambda i: (i, 0)),
            pl.BlockSpec((2, ROW_BLK, D), lambda i: (0, i, 0)),
            pl.BlockSpec((D, D), lambda i: (0, 0)),
            pl.BlockSpec((1, D), lambda i: (0, 0)),
        ],
        out_specs=pl.BlockSpec((ROW_BLK, D), lambda i: (i, 0)),
        out_shape=jax.ShapeDtypeStruct((NP, D), jnp.float32),
    )(f_pad, hp, W2_w, W2_b.reshape(1, D))


@jax.jit
def _run(indices, features, W1_w, W1_b, W2_w, W2_b, Watt_w, Watt_b, a):
    f_pad = jnp.pad(features, ((0, NP - N), (0, 0)))
    idx_pad = jnp.pad(indices, ((0, 0), (0, EP - E)), constant_values=N)
    aT = a.reshape(2, D)

    msg, pq = _k1(f_pad, W1_w, W1_b, Watt_w, Watt_b, aT)
    e_edge, den = _k2(idx_pad, pq[0], pq[1])
    msg2 = _k3(msg, den)
    hp = _k4(idx_pad, e_edge, msg2)
    out = _k5(f_pad, hp, W2_w, W2_b)
    return out[:N]


def kernel(indices, features, num_nodes, W1_w, W1_b, W2_w, W2_b, Watt_w, Watt_b, a):
    return _run(indices, features, W1_w, W1_b, W2_w, W2_b, Watt_w, Watt_b, a)


# speedup vs baseline: 1.2678x; 1.0033x over previous
import functools

import jax
import jax.numpy as jnp
from jax import lax
from jax.experimental import pallas as pl
from jax.experimental.pallas import tpu as pltpu
from jax.experimental.pallas import tpu_sc as plsc

N = 10000
D = 128
E = 320000

NUM_TILES = 16
NUM_CORES = 2
NW = NUM_TILES * NUM_CORES
C = 128

NP = 10240
EP = ((E + NW * C - 1) // (NW * C)) * (NW * C)
EDGES_PER_TILE = EP // NW
CHUNKS = EDGES_PER_TILE // C
STRIPE = NP // NUM_TILES

ROW_BLK = 1024
TC_GRID = NP // ROW_BLK


def _k1_body(f_ref, w1_ref, b1_ref, watt_ref, battr_ref, at_ref, msg_ref, pq_ref):
    fb = f_ref[...]
    msg_ref[...] = (
        jnp.dot(fb, w1_ref[...].T, preferred_element_type=jnp.float32) + b1_ref[...]
    )
    v = jnp.dot(at_ref[...], watt_ref[...], preferred_element_type=jnp.float32)
    c = jnp.dot(at_ref[...], battr_ref[...].T, preferred_element_type=jnp.float32)
    pq_ref[...] = jnp.dot(v, fb.T, preferred_element_type=jnp.float32) + c


def _k1(f_pad, W1_w, W1_b, Watt_w, Watt_b, aT):
    return pl.pallas_call(
        _k1_body,
        grid=(TC_GRID,),
        in_specs=[
            pl.BlockSpec((ROW_BLK, D), lambda i: (i, 0)),
            pl.BlockSpec((D, D), lambda i: (0, 0)),
            pl.BlockSpec((1, D), lambda i: (0, 0)),
            pl.BlockSpec((D, D), lambda i: (0, 0)),
            pl.BlockSpec((1, D), lambda i: (0, 0)),
            pl.BlockSpec((2, D), lambda i: (0, 0)),
        ],
        out_specs=[
            pl.BlockSpec((ROW_BLK, D), lambda i: (i, 0)),
            pl.BlockSpec((2, ROW_BLK), lambda i: (0, i)),
        ],
        out_shape=[
            jax.ShapeDtypeStruct((NP, D), jnp.float32),
            jax.ShapeDtypeStruct((2, NP), jnp.float32),
        ],
    )(f_pad, W1_w, W1_b.reshape(1, D), Watt_w, Watt_b.reshape(1, D), aT)


def _k2_body(idx_hbm, p_hbm, q_hbm, e_hbm, den_hbm, iv, pg, qg, ev, zb, den_sh, sem):
    cid = lax.axis_index("c")
    sid = lax.axis_index("s")
    wid = sid * NUM_CORES + cid

    def _z(i, carry):
        zb[pl.ds(i * 16, 16)] = jnp.zeros((16,), jnp.float32)
        return carry
    lax.fori_loop(0, STRIPE // 16, _z, 0)
    pltpu.sync_copy(zb, den_sh.at[pl.ds(sid * STRIPE, STRIPE)])
    plsc.subcore_barrier()

    base = wid * EDGES_PER_TILE

    def _chunk(t, carry):
        off = base + t * C
        pltpu.sync_copy(idx_hbm.at[:, pl.ds(off, C)], iv)
        cp1 = pltpu.async_copy(p_hbm.at[iv.at[0]], pg, sem)
        cp2 = pltpu.async_copy(q_hbm.at[iv.at[1]], qg, sem)
        cp1.wait()
        cp2.wait()
        for j in range(C // 16):
            x = pg[pl.ds(j * 16, 16)] + qg[pl.ds(j * 16, 16)]
            x = jnp.where(x >= 0.0, x, x * 0.01)
            ev[pl.ds(j * 16, 16)] = jnp.exp(x)
        pltpu.sync_copy(ev, e_hbm.at[pl.ds(off, C)])
        pltpu.sync_copy(ev, den_sh.at[iv.at[1]], add=True)
        return carry

    lax.fori_loop(0, CHUNKS, _chunk, 0)
    plsc.subcore_barrier()

    pltpu.sync_copy(den_sh.at[pl.ds(sid * STRIPE, STRIPE)], zb)
    pltpu.sync_copy(zb, den_hbm.at[cid, pl.ds(sid * STRIPE, STRIPE)])


def _k2(idx_pad, p, q):
    mesh = plsc.VectorSubcoreMesh(core_axis_name="c", subcore_axis_name="s")
    return pl.kernel(
        _k2_body,
        mesh=mesh,
        out_type=[
            jax.ShapeDtypeStruct((EP,), jnp.float32),
            jax.ShapeDtypeStruct((2, NP), jnp.float32),
        ],
        scratch_types=[
            pltpu.VMEM((2, C), jnp.int32),
            pltpu.VMEM((C,), jnp.float32),
            pltpu.VMEM((C,), jnp.float32),
            pltpu.VMEM((C,), jnp.float32),
            pltpu.VMEM((STRIPE,), jnp.float32),
            pltpu.VMEM_SHARED((NP,), jnp.float32),
            pltpu.SemaphoreType.DMA,
        ],
    )(idx_pad, p, q)


def _k3_body(msg_ref, den_ref, out_ref):
    d = den_ref[0, :] + den_ref[1, :] + 1e-9
    out_ref[...] = msg_ref[...] / d[:, None]


def _k3(msg, den):
    return pl.pallas_call(
        _k3_body,
        grid=(TC_GRID,),
        in_specs=[
            pl.BlockSpec((ROW_BLK, D), lambda i: (i, 0)),
            pl.BlockSpec((2, ROW_BLK), lambda i: (0, i)),
        ],
        out_specs=pl.BlockSpec((ROW_BLK, D), lambda i: (i, 0)),
        out_shape=jax.ShapeDtypeStruct((NP, D), jnp.float32),
    )(msg, den)


def _k4_body(idx_hbm, e_hbm, msg2_hbm, hp_hbm, iv, ev, rows, zb, acc_sh, sem):
    cid = lax.axis_index("c")
    sid = lax.axis_index("s")
    wid = sid * NUM_CORES + cid

    def _z(i, carry):
        for j in range(D // 16):
            zb[i, pl.ds(j * 16, 16)] = jnp.zeros((16,), jnp.float32)
        return carry
    lax.fori_loop(0, C, _z, 0)
    for r in range(STRIPE // C):
        pltpu.sync_copy(zb, acc_sh.at[pl.ds(sid * STRIPE + r * C, C), :])
    plsc.subcore_barrier()

    base = wid * EDGES_PER_TILE

    def _chunk(t, carry):
        off = base + t * C
        pltpu.sync_copy(idx_hbm.at[:, pl.ds(off, C)], iv)
        pltpu.sync_copy(e_hbm.at[pl.ds(off, C)], ev)
        pltpu.async_copy(msg2_hbm.at[iv.at[1]], rows, sem).wait()

        @plsc.parallel_loop(0, C // 16, unroll=2)
        def _scale(g):
            ev16 = ev[pl.ds(g * 16, 16)]
            for l in range(16):
                e = ev16[l]
                i = g * 16 + l
                for j in range(D // 16):
                    rows[i, pl.ds(j * 16, 16)] = rows[i, pl.ds(j * 16, 16)] * e
        pltpu.sync_copy(rows, acc_sh.at[iv.at[0]], add=True)
        return carry

    lax.fori_loop(0, CHUNKS, _chunk, 0)
    plsc.subcore_barrier()

    for r in range(STRIPE // C):
        pltpu.sync_copy(acc_sh.at[pl.ds(sid * STRIPE + r * C, C), :], zb)
        pltpu.sync_copy(zb, hp_hbm.at[cid, pl.ds(sid * STRIPE + r * C, C), :])


def _k4(idx_pad, e_edge, msg2):
    mesh = plsc.VectorSubcoreMesh(core_axis_name="c", subcore_axis_name="s")
    return pl.kernel(
        _k4_body,
        mesh=mesh,
        out_type=jax.ShapeDtypeStruct((2, NP, D), jnp.float32),
        scratch_types=[
            pltpu.VMEM((2, C), jnp.int32),
            pltpu.VMEM((C,), jnp.float32),
            pltpu.VMEM((C, D), jnp.float32),
            pltpu.VMEM((C, D), jnp.float32),
            pltpu.VMEM_SHARED((NP, D), jnp.float32),
            pltpu.SemaphoreType.DMA,
        ],
    )(idx_pad, e_edge, msg2)


def _k5_body(f_ref, hp_ref, w2_ref, b2_ref, out_ref):
    fb = f_ref[...]
    hn = hp_ref[0] + hp_ref[1]
    t2 = jnp.dot(fb * hn, w2_ref[...].T, preferred_element_type=jnp.float32) + b2_ref[...]
    o = fb + hn + t2
    out_ref[...] = jnp.where(o >= 0.0, o, o * 0.01)


def _k5(f_pad, hp, W2_w, W2_b):
    return pl.pallas_call(
        _k5_body,
        grid=(TC_GRID,),
        in_specs=[
            pl.BlockSpec((ROW_BLK, D), lambda i: (i, 0)),
            pl.BlockSpec((2, ROW_BLK, D), lambda i: (0, i, 0)),
            pl.BlockSpec((D, D), lambda i: (0, 0)),
            pl.BlockSpec((1, D), lambda i: (0, 0)),
        ],
        out_specs=pl.BlockSpec((ROW_BLK, D), lambda i: (i, 0)),
        out_shape=jax.ShapeDtypeStruct((NP, D), jnp.float32),
    )(f_pad, hp, W2_w, W2_b.reshape(1, D))


@jax.jit
def _run(indices, features, W1_w, W1_b, W2_w, W2_b, Watt_w, Watt_b, a):
    f_pad = jnp.pad(features, ((0, NP - N), (0, 0)))
    idx_pad = jnp.pad(indices, ((0, 0), (0, EP - E)), constant_values=N)
    aT = a.reshape(2, D)

    msg, pq = _k1(f_pad, W1_w, W1_b, Watt_w, Watt_b, aT)
    e_edge, den = _k2(idx_pad, pq[0], pq[1])
    msg2 = _k3(msg, den)
    hp = _k4(idx_pad, e_edge, msg2)
    out = _k5(f_pad, hp, W2_w, W2_b)
    return out[:N]


def kernel(indices, features, num_nodes, W1_w, W1_b, W2_w, W2_b, Watt_w, Watt_b, a):
    return _run(indices, features, W1_w, W1_b, W2_w, W2_b, Watt_w, Watt_b, a)
